# baseline (speedup 1.0000x reference)
import functools

import jax
import jax.numpy as jnp
from jax import lax
from jax.experimental import pallas as pl
from jax.experimental.pallas import tpu as pltpu
from jax.experimental.pallas import tpu_sc as plsc

N_LAYERS = 4
EPS_FA = 1.0
LN_EPS = 1e-5

_BLK = 1024
_NW = 32
_CH = 128
_D = 128

_NP = 10240
_N_SH = 10016
_ROWS_PER_TILE = _N_SH // 16
_DUMMY = 10000


def _elu(x):
    return jnp.where(x > 0, x, jnp.exp(jnp.minimum(x, 0.0)) - 1.0)


def _tc_prep_body(histp_ref, mask_ref, x_ref, attl_ref, attr_ref,
                  dis_ref, al_ref, ar_ref, hd_ref):
    deg = jnp.sum(histp_ref[...], axis=0) + 1.0
    dis = mask_ref[...] * jax.lax.rsqrt(deg)
    dis_ref[...] = dis
    x = x_ref[...]
    al_ref[...] = jnp.sum(x * attl_ref[...][None, :], axis=1)
    ar_ref[...] = jnp.sum(x * attr_ref[...][None, :], axis=1)
    hd_ref[...] = dis[:, None] * x


def _tc_prep(histp, mask, x, attl, attr):
    np_, d = x.shape
    grid = np_ // _BLK
    v1 = pl.BlockSpec((_BLK,), lambda i: (i,))
    vfull = pl.BlockSpec((d,), lambda i: (0,))
    m2 = pl.BlockSpec((_BLK, d), lambda i: (i, 0))
    hp = pl.BlockSpec((_NW, _BLK), lambda i: (0, i))
    return pl.pallas_call(
        _tc_prep_body,
        grid=(grid,),
        in_specs=[hp, v1, m2, vfull, vfull],
        out_specs=[v1, v1, v1, m2],
        out_shape=[jax.ShapeDtypeStruct((np_,), jnp.float32),
                   jax.ShapeDtypeStruct((np_,), jnp.float32),
                   jax.ShapeDtypeStruct((np_,), jnp.float32),
                   jax.ShapeDtypeStruct((np_, d), jnp.float32)],
    )(histp, mask, x, attl, attr)


def _tc_dense_body(p0_ref, p1_ref, h_ref, h0_ref, al_ref, ar_ref, dis_ref,
                   g_ref, b_ref, attl_ref, attr_ref,
                   hn_ref, aln_ref, arn_ref, hd_ref):
    dis = dis_ref[...]
    selfc = jnp.tanh(al_ref[...] + ar_ref[...]) * dis * dis
    agg = (dis[:, None] * (p0_ref[...] + p1_ref[...])
           + selfc[:, None] * h_ref[...] + EPS_FA * h0_ref[...])
    a = _elu(agg)
    mu = jnp.mean(a, axis=1, keepdims=True)
    var = jnp.mean((a - mu) ** 2, axis=1, keepdims=True)
    hn = (a - mu) * jax.lax.rsqrt(var + LN_EPS) * g_ref[...][None, :] \
        + b_ref[...][None, :]
    hn_ref[...] = hn
    aln_ref[...] = jnp.sum(hn * attl_ref[...][None, :], axis=1)
    arn_ref[...] = jnp.sum(hn * attr_ref[...][None, :], axis=1)
    hd_ref[...] = dis[:, None] * hn


def _tc_dense(p0, p1, h, h0, al, ar, dis, gamma, beta, attl, attr):
    np_, d = h.shape
    grid = np_ // _BLK
    v1 = pl.BlockSpec((_BLK,), lambda i: (i,))
    vfull = pl.BlockSpec((d,), lambda i: (0,))
    m2 = pl.BlockSpec((_BLK, d), lambda i: (i, 0))
    return pl.pallas_call(
        _tc_dense_body,
        grid=(grid,),
        in_specs=[m2, m2, m2, m2, v1, v1, v1, vfull, vfull, vfull, vfull],
        out_specs=[m2, v1, v1, m2],
        out_shape=[jax.ShapeDtypeStruct((np_, d), jnp.float32),
                   jax.ShapeDtypeStruct((np_,), jnp.float32),
                   jax.ShapeDtypeStruct((np_,), jnp.float32),
                   jax.ShapeDtypeStruct((np_, d), jnp.float32)],
    )(p0, p1, h, h0, al, ar, dis, gamma, beta, attl, attr)


def _tc_final_body(p0_ref, p1_ref, h_ref, h0_ref, al_ref, ar_ref, dis_ref,
                   g_ref, b_ref, w_ref, bias_ref, out_ref):
    dis = dis_ref[...]
    selfc = jnp.tanh(al_ref[...] + ar_ref[...]) * dis * dis
    agg = (dis[:, None] * (p0_ref[...] + p1_ref[...])
           + selfc[:, None] * h_ref[...] + EPS_FA * h0_ref[...])
    a = _elu(agg)
    mu = jnp.mean(a, axis=1, keepdims=True)
    var = jnp.mean((a - mu) ** 2, axis=1, keepdims=True)
    hn = (a - mu) * jax.lax.rsqrt(var + LN_EPS) * g_ref[...][None, :] \
        + b_ref[...][None, :]
    out_ref[...] = jax.lax.dot_general(
        hn, w_ref[...], (((1,), (1,)), ((), ())),
        preferred_element_type=jnp.float32) + bias_ref[...][None, :]


def _tc_final(p0, p1, h, h0, al, ar, dis, gamma, beta, w, bias):
    np_, d = h.shape
    c = w.shape[0]
    grid = np_ // _BLK
    v1 = pl.BlockSpec((_BLK,), lambda i: (i,))
    vfull = pl.BlockSpec((d,), lambda i: (0,))
    m2 = pl.BlockSpec((_BLK, d), lambda i: (i, 0))
    wspec = pl.BlockSpec((c, d), lambda i: (0, 0))
    bspec = pl.BlockSpec((c,), lambda i: (0,))
    return pl.pallas_call(
        _tc_final_body,
        grid=(grid,),
        in_specs=[m2, m2, m2, m2, v1, v1, v1, vfull, vfull, wspec, bspec],
        out_specs=pl.BlockSpec((_BLK, c), lambda i: (i, 0)),
        out_shape=jax.ShapeDtypeStruct((np_, c), jnp.float32),
    )(p0, p1, h, h0, al, ar, dis, gamma, beta, w, bias)


_MESH = plsc.VectorSubcoreMesh(core_axis_name="c", subcore_axis_name="s")
_SC_PARAMS = pltpu.CompilerParams(
    needs_layout_passes=False, use_tc_tiling_on_sc=False)


def _sc_deg_body(rc_hbm, deg_hbm, rc_v, deg_v):
    nch = rc_hbm.shape[1]
    cid = lax.axis_index("c")
    tid = lax.axis_index("s")
    wid = cid * 16 + tid
    pltpu.sync_copy(rc_hbm.at[wid], rc_v)

    zero16 = jnp.zeros((16,), jnp.float32)

    def zrow(j, _):
        deg_v[pl.ds(j * 16, 16)] = zero16
        return 0

    lax.fori_loop(0, deg_v.shape[0] // 16, zrow, 0)

    ones16 = jnp.ones((16,), jnp.float32)

    def chunk(ch, _):
        for g in range(_CH // 16):
            c = rc_v[ch, 1, pl.ds(g * 16, 16)]
            plsc.addupdate_scatter(deg_v, [c], ones16)
        return 0

    lax.fori_loop(0, nch, chunk, 0)
    pltpu.sync_copy(deg_v, deg_hbm.at[wid])


def _sc_deg(rc):
    nch = rc.shape[1]
    f = pl.kernel(
        _sc_deg_body,
        out_type=jax.ShapeDtypeStruct((_NW, _N_SH), jnp.float32),
        mesh=_MESH,
        scratch_types=[
            pltpu.VMEM((nch, 2, _CH), jnp.int32),
            pltpu.VMEM((_N_SH,), jnp.float32),
        ],
        compiler_params=_SC_PARAMS,
    )
    return f(rc)


def _sc_coef_body(rc_hbm, al_hbm, ar_hbm, coef_hbm, rc_v, al_v, ar_v,
                  coef_v):
    nch = rc_hbm.shape[1]
    cid = lax.axis_index("c")
    tid = lax.axis_index("s")
    wid = cid * 16 + tid
    pltpu.sync_copy(rc_hbm.at[wid], rc_v)
    pltpu.sync_copy(al_hbm, al_v)
    pltpu.sync_copy(ar_hbm, ar_v)

    def chunk(ch, _):
        for g in range(_CH // 16):
            sl = pl.ds(g * 16, 16)
            r = rc_v[ch, 0, sl]
            c = rc_v[ch, 1, sl]
            s = plsc.load_gather(al_v, [r]) + plsc.load_gather(ar_v, [c])
            e = jnp.exp(-2.0 * jnp.abs(s))
            t = (1.0 - e) / (1.0 + e)
            coef_v[ch, sl] = jnp.where(s < 0, -t, t)
        return 0

    lax.fori_loop(0, nch, chunk, 0)
    pltpu.sync_copy(coef_v, coef_hbm.at[wid])


def _sc_coef(rc, al_p, ar_p):
    nch = rc.shape[1]
    f = pl.kernel(
        _sc_coef_body,
        out_type=jax.ShapeDtypeStruct((_NW, nch, _CH), jnp.float32),
        mesh=_MESH,
        scratch_types=[
            pltpu.VMEM((nch, 2, _CH), jnp.int32),
            pltpu.VMEM((_N_SH,), jnp.float32),
            pltpu.VMEM((_N_SH,), jnp.float32),
            pltpu.VMEM((nch, _CH), jnp.float32),
        ],
        compiler_params=_SC_PARAMS,
    )
    return f(rc, al_p, ar_p)


def _sc_edge_body(hd_hbm, rc_hbm, cf_hbm, part_hbm,
                  idx0, idx1, idx2, cfb0, cfb1, cfb2, sci0, sci1, sci2,
                  rows0, rows1, rows2, agg_sh,
                  semi0, semi1, semi2, semg0, semg1, semg2,
                  sems0, sems1, sems2):
    nch = rc_hbm.shape[1]
    cid = lax.axis_index("c")
    tid = lax.axis_index("s")
    wid = cid * 16 + tid
    base = tid * _ROWS_PER_TILE

    idxv = (idx0, idx1, idx2)
    cfv = (cfb0, cfb1, cfb2)
    sciv = (sci0, sci1, sci2)
    rowsv = (rows0, rows1, rows2)
    semi = (semi0, semi1, semi2)
    semg = (semg0, semg1, semg2)
    sems = (sems0, sems1, sems2)

    zero16 = jnp.zeros((16,), jnp.float32)

    @plsc.parallel_loop(0, _CH)
    def _(j):
        for g in range(8):
            rows0[j, pl.ds(g * 16, 16)] = zero16

    nfull = _ROWS_PER_TILE // _CH
    rem = _ROWS_PER_TILE - nfull * _CH
    for k in range(nfull):
        pltpu.sync_copy(rows0, agg_sh.at[pl.ds(base + k * _CH, _CH)])
    if rem:
        pltpu.sync_copy(rows0.at[pl.ds(0, rem)],
                        agg_sh.at[pl.ds(base + nfull * _CH, rem)])
    plsc.subcore_barrier()

    def start_idx(c, b):
        d1 = pltpu.async_copy(rc_hbm.at[wid, c], idxv[b], semi[b])
        d2 = pltpu.async_copy(cf_hbm.at[wid, c], cfv[b], semi[b])
        return d1, d2

    def wait_idx(b):
        pltpu.make_async_copy(rc_hbm.at[wid, 0], idxv[b], semi[b]).wait()
        pltpu.make_async_copy(cf_hbm.at[wid, 0], cfv[b], semi[b]).wait()

    def start_gather(b):
        return pltpu.async_copy(hd_hbm.at[idxv[b].at[0]], rowsv[b], semg[b])

    def wait_gather(b):
        pltpu.make_async_copy(hd_hbm.at[idxv[b].at[0]], rowsv[b],
                              semg[b]).wait()

    def start_scatter(b):
        return pltpu.async_copy(rowsv[b], agg_sh.at[sciv[b].at[0]],
                                sems[b], add=True)

    def wait_scatter(b):
        pltpu.make_async_copy(rowsv[b], agg_sh.at[sciv[b].at[0]],
                              sems[b]).wait()

    def scale(b):
        @plsc.parallel_loop(0, _CH, unroll=4)
        def _(j):
            cj = plsc.load_gather(
                cfv[b],
                [jnp.broadcast_to(j, (16,)).astype(jnp.int32)])
            for g in range(8):
                sl = pl.ds(g * 16, 16)
                rowsv[b][j, sl] = rowsv[b][j, sl] * cj

    def copy_sci(b):
        for g in range(_CH // 16):
            sl = pl.ds(g * 16, 16)
            sciv[b][0, sl] = idxv[b][1, sl]

    def step(c, b, wait_s=True, start_g=True, start_i=True):
        b1 = (b + 1) % 3
        if wait_s:
            wait_scatter(b1)
        if start_g:
            wait_idx(b1)
            start_gather(b1)
        wait_gather(b)
        scale(b)
        copy_sci(b)
        start_scatter(b)
        if start_i:
            start_idx(c + 3, b)

    d = start_idx(0, 0)
    d[0].wait()
    d[1].wait()
    start_gather(0)
    start_idx(1, 1)
    start_idx(2, 2)
    step(0, 0, wait_s=False)
    step(1, 1, wait_s=False)

    def triple(k, _):
        c0 = 3 * k + 2
        step(c0, 2)
        step(c0 + 1, 0)
        step(c0 + 2, 1)
        return 0

    nk = (nch - 6) // 3
    lax.fori_loop(0, nk, triple, 0)

    cA = nch - 4
    step(cA, cA % 3)
    step(cA + 1, (cA + 1) % 3, start_i=False)
    step(cA + 2, (cA + 2) % 3, start_i=False)
    step(cA + 3, (cA + 3) % 3, start_g=False, start_i=False)
    wait_scatter((cA + 2) % 3)
    wait_scatter((cA + 3) % 3)

    plsc.subcore_barrier()
    pltpu.sync_copy(agg_sh.at[pl.ds(base, _ROWS_PER_TILE)],
                    part_hbm.at[cid, pl.ds(base, _ROWS_PER_TILE)])


def _sc_edge(hd, rc, cf):
    nch = rc.shape[1]
    f = pl.kernel(
        _sc_edge_body,
        out_type=jax.ShapeDtypeStruct((2, _N_SH, _D), jnp.float32),
        mesh=_MESH,
        scratch_types=[
            pltpu.VMEM((2, _CH), jnp.int32),
            pltpu.VMEM((2, _CH), jnp.int32),
            pltpu.VMEM((2, _CH), jnp.int32),
            pltpu.VMEM((_CH,), jnp.float32),
            pltpu.VMEM((_CH,), jnp.float32),
            pltpu.VMEM((_CH,), jnp.float32),
            pltpu.VMEM((1, _CH), jnp.int32),
            pltpu.VMEM((1, _CH), jnp.int32),
            pltpu.VMEM((1, _CH), jnp.int32),
            pltpu.VMEM((_CH, _D), jnp.float32),
            pltpu.VMEM((_CH, _D), jnp.float32),
            pltpu.VMEM((_CH, _D), jnp.float32),
            pltpu.VMEM_SHARED((_N_SH, _D), jnp.float32),
            pltpu.SemaphoreType.DMA,
            pltpu.SemaphoreType.DMA,
            pltpu.SemaphoreType.DMA,
            pltpu.SemaphoreType.DMA,
            pltpu.SemaphoreType.DMA,
            pltpu.SemaphoreType.DMA,
            pltpu.SemaphoreType.DMA,
            pltpu.SemaphoreType.DMA,
            pltpu.SemaphoreType.DMA,
        ],
        compiler_params=_SC_PARAMS,
    )
    return f(hd, rc, cf)


def kernel(x, edge_index, att_l, att_r, ln_gamma, ln_beta, W, b):
    n, d = x.shape
    e = edge_index.shape[1]
    row = edge_index[0]
    col = edge_index[1]
    xp = jnp.pad(x, ((0, _NP - n), (0, 0)))
    mask = (jnp.arange(_NP) < n).astype(jnp.float32)

    epw = e // _NW
    nch = (epw + _CH - 1) // _CH
    nch = ((nch + 2) // 3) * 3
    pad = nch * _CH - epw
    row3 = jnp.pad(row.reshape(_NW, epw), ((0, 0), (0, pad))) \
        .reshape(_NW, nch, 1, _CH)
    col3 = jnp.pad(col.reshape(_NW, epw), ((0, 0), (0, pad)),
                   constant_values=_DUMMY).reshape(_NW, nch, 1, _CH)
    rc = jnp.concatenate([row3, col3], axis=2)

    degp = _sc_deg(rc)
    histp = jnp.pad(degp, ((0, 0), (0, _NP - _N_SH)))
    dis, al, ar, hd = _tc_prep(histp, mask, xp, att_l[0], att_r[0])

    h = xp
    for layer in range(N_LAYERS):
        cf = _sc_coef(rc, al[:_N_SH], ar[:_N_SH])
        part = _sc_edge(hd, rc, cf)
        p0 = jnp.pad(part[0], ((0, _NP - _N_SH), (0, 0)))
        p1 = jnp.pad(part[1], ((0, _NP - _N_SH), (0, 0)))
        if layer < N_LAYERS - 1:
            h, al, ar, hd = _tc_dense(p0, p1, h, xp, al, ar, dis,
                                      ln_gamma[layer], ln_beta[layer],
                                      att_l[layer + 1], att_r[layer + 1])
        else:
            out = _tc_final(p0, p1, h, xp, al, ar, dis,
                            ln_gamma[layer], ln_beta[layer], W, b)
    return out[:n]


# speedup vs baseline: 2.1986x; 2.1986x over previous
import functools

import jax
import jax.numpy as jnp
from jax import lax
from jax.experimental import pallas as pl
from jax.experimental.pallas import tpu as pltpu
from jax.experimental.pallas import tpu_sc as plsc

N_LAYERS = 4
EPS_FA = 1.0
LN_EPS = 1e-5

_BLK = 1024
_NW = 32
_CH = 112
_D = 128

_NP = 10240
_N_SH = 10016
_ROWS_PER_TILE = _N_SH // 16
_DUMMY = 10000


def _elu(x):
    return jnp.where(x > 0, x, jnp.exp(jnp.minimum(x, 0.0)) - 1.0)


def _tc_prep_body(histp_ref, mask_ref, x_ref, attl_ref, attr_ref,
                  dis_ref, al_ref, ar_ref, hd_ref):
    deg = jnp.sum(histp_ref[...], axis=0) + 1.0
    dis = mask_ref[...] * jax.lax.rsqrt(deg)
    dis_ref[...] = dis
    x = x_ref[...]
    al_ref[...] = jnp.sum(x * attl_ref[...][None, :], axis=1)
    ar_ref[...] = jnp.sum(x * attr_ref[...][None, :], axis=1)
    hd_ref[...] = dis[:, None] * x


def _tc_prep(histp, mask, x, attl, attr):
    np_, d = x.shape
    grid = np_ // _BLK
    v1 = pl.BlockSpec((_BLK,), lambda i: (i,))
    vfull = pl.BlockSpec((d,), lambda i: (0,))
    m2 = pl.BlockSpec((_BLK, d), lambda i: (i, 0))
    hp = pl.BlockSpec((_NW, _BLK), lambda i: (0, i))
    return pl.pallas_call(
        _tc_prep_body,
        grid=(grid,),
        in_specs=[hp, v1, m2, vfull, vfull],
        out_specs=[v1, v1, v1, m2],
        out_shape=[jax.ShapeDtypeStruct((np_,), jnp.float32),
                   jax.ShapeDtypeStruct((np_,), jnp.float32),
                   jax.ShapeDtypeStruct((np_,), jnp.float32),
                   jax.ShapeDtypeStruct((np_, d), jnp.float32)],
    )(histp, mask, x, attl, attr)


def _tc_dense_body(p0_ref, p1_ref, h_ref, h0_ref, al_ref, ar_ref, dis_ref,
                   g_ref, b_ref, attl_ref, attr_ref,
                   hn_ref, aln_ref, arn_ref, hd_ref):
    dis = dis_ref[...]
    selfc = jnp.tanh(al_ref[...] + ar_ref[...]) * dis * dis
    agg = (dis[:, None] * (p0_ref[...] + p1_ref[...])
           + selfc[:, None] * h_ref[...] + EPS_FA * h0_ref[...])
    a = _elu(agg)
    mu = jnp.mean(a, axis=1, keepdims=True)
    var = jnp.mean((a - mu) ** 2, axis=1, keepdims=True)
    hn = (a - mu) * jax.lax.rsqrt(var + LN_EPS) * g_ref[...][None, :] \
        + b_ref[...][None, :]
    hn_ref[...] = hn
    aln_ref[...] = jnp.sum(hn * attl_ref[...][None, :], axis=1)
    arn_ref[...] = jnp.sum(hn * attr_ref[...][None, :], axis=1)
    hd_ref[...] = dis[:, None] * hn


def _tc_dense(p0, p1, h, h0, al, ar, dis, gamma, beta, attl, attr):
    np_, d = h.shape
    grid = np_ // _BLK
    v1 = pl.BlockSpec((_BLK,), lambda i: (i,))
    vfull = pl.BlockSpec((d,), lambda i: (0,))
    m2 = pl.BlockSpec((_BLK, d), lambda i: (i, 0))
    return pl.pallas_call(
        _tc_dense_body,
        grid=(grid,),
        in_specs=[m2, m2, m2, m2, v1, v1, v1, vfull, vfull, vfull, vfull],
        out_specs=[m2, v1, v1, m2],
        out_shape=[jax.ShapeDtypeStruct((np_, d), jnp.float32),
                   jax.ShapeDtypeStruct((np_,), jnp.float32),
                   jax.ShapeDtypeStruct((np_,), jnp.float32),
                   jax.ShapeDtypeStruct((np_, d), jnp.float32)],
    )(p0, p1, h, h0, al, ar, dis, gamma, beta, attl, attr)


def _tc_final_body(p0_ref, p1_ref, h_ref, h0_ref, al_ref, ar_ref, dis_ref,
                   g_ref, b_ref, w_ref, bias_ref, out_ref):
    dis = dis_ref[...]
    selfc = jnp.tanh(al_ref[...] + ar_ref[...]) * dis * dis
    agg = (dis[:, None] * (p0_ref[...] + p1_ref[...])
           + selfc[:, None] * h_ref[...] + EPS_FA * h0_ref[...])
    a = _elu(agg)
    mu = jnp.mean(a, axis=1, keepdims=True)
    var = jnp.mean((a - mu) ** 2, axis=1, keepdims=True)
    hn = (a - mu) * jax.lax.rsqrt(var + LN_EPS) * g_ref[...][None, :] \
        + b_ref[...][None, :]
    out_ref[...] = jax.lax.dot_general(
        hn, w_ref[...], (((1,), (1,)), ((), ())),
        preferred_element_type=jnp.float32) + bias_ref[...][None, :]


def _tc_final(p0, p1, h, h0, al, ar, dis, gamma, beta, w, bias):
    np_, d = h.shape
    c = w.shape[0]
    grid = np_ // _BLK
    v1 = pl.BlockSpec((_BLK,), lambda i: (i,))
    vfull = pl.BlockSpec((d,), lambda i: (0,))
    m2 = pl.BlockSpec((_BLK, d), lambda i: (i, 0))
    wspec = pl.BlockSpec((c, d), lambda i: (0, 0))
    bspec = pl.BlockSpec((c,), lambda i: (0,))
    return pl.pallas_call(
        _tc_final_body,
        grid=(grid,),
        in_specs=[m2, m2, m2, m2, v1, v1, v1, vfull, vfull, wspec, bspec],
        out_specs=pl.BlockSpec((_BLK, c), lambda i: (i, 0)),
        out_shape=jax.ShapeDtypeStruct((np_, c), jnp.float32),
    )(p0, p1, h, h0, al, ar, dis, gamma, beta, w, bias)


_MESH = plsc.VectorSubcoreMesh(core_axis_name="c", subcore_axis_name="s")
_SC_PARAMS = pltpu.CompilerParams(
    needs_layout_passes=False, use_tc_tiling_on_sc=False)


def _sc_deg_body(rc_hbm, deg_hbm, rc_v, deg_v):
    nch = rc_hbm.shape[1]
    cid = lax.axis_index("c")
    tid = lax.axis_index("s")
    wid = cid * 16 + tid
    pltpu.sync_copy(rc_hbm.at[wid], rc_v)

    zero16 = jnp.zeros((16,), jnp.float32)

    def zrow(j, _):
        deg_v[pl.ds(j * 16, 16)] = zero16
        return 0

    lax.fori_loop(0, deg_v.shape[0] // 16, zrow, 0)

    ones16 = jnp.ones((16,), jnp.float32)

    def chunk(ch, _):
        for g in range(_CH // 16):
            c = rc_v[ch, 1, pl.ds(g * 16, 16)]
            plsc.addupdate_scatter(deg_v, [c], ones16)
        return 0

    lax.fori_loop(0, nch, chunk, 0)
    pltpu.sync_copy(deg_v, deg_hbm.at[wid])


def _sc_deg(rc):
    nch = rc.shape[1]
    f = pl.kernel(
        _sc_deg_body,
        out_type=jax.ShapeDtypeStruct((_NW, _N_SH), jnp.float32),
        mesh=_MESH,
        scratch_types=[
            pltpu.VMEM((nch, 2, _CH), jnp.int32),
            pltpu.VMEM((_N_SH,), jnp.float32),
        ],
        compiler_params=_SC_PARAMS,
    )
    return f(rc)


def _sc_coef_body(rc_hbm, al_hbm, ar_hbm, coef_hbm, rc_v, al_v, ar_v,
                  coef_v):
    nch = rc_hbm.shape[1]
    cid = lax.axis_index("c")
    tid = lax.axis_index("s")
    wid = cid * 16 + tid
    pltpu.sync_copy(rc_hbm.at[wid], rc_v)
    pltpu.sync_copy(al_hbm, al_v)
    pltpu.sync_copy(ar_hbm, ar_v)

    def chunk(ch, _):
        for g in range(_CH // 16):
            sl = pl.ds(g * 16, 16)
            r = rc_v[ch, 0, sl]
            c = rc_v[ch, 1, sl]
            s = plsc.load_gather(al_v, [r]) + plsc.load_gather(ar_v, [c])
            e = jnp.exp(-2.0 * jnp.abs(s))
            t = (1.0 - e) / (1.0 + e)
            coef_v[ch, sl] = jnp.where(s < 0, -t, t)
        return 0

    lax.fori_loop(0, nch, chunk, 0)
    pltpu.sync_copy(coef_v, coef_hbm.at[wid])


def _sc_coef(rc, al_p, ar_p):
    nch = rc.shape[1]
    f = pl.kernel(
        _sc_coef_body,
        out_type=jax.ShapeDtypeStruct((_NW, nch, _CH), jnp.float32),
        mesh=_MESH,
        scratch_types=[
            pltpu.VMEM((nch, 2, _CH), jnp.int32),
            pltpu.VMEM((_N_SH,), jnp.float32),
            pltpu.VMEM((_N_SH,), jnp.float32),
            pltpu.VMEM((nch, _CH), jnp.float32),
        ],
        compiler_params=_SC_PARAMS,
    )
    return f(rc, al_p, ar_p)


def _sc_edge_body(hd_hbm, rc_hbm, cf_hbm, part_hbm,
                  idx0, idx1, idx2, cfb0, cfb1, cfb2, sci0, sci1, sci2,
                  rows0, rows1, rows2, agg_sh,
                  semi0, semi1, semi2, semg0, semg1, semg2,
                  sems0, sems1, sems2):
    nch = rc_hbm.shape[1]
    cid = lax.axis_index("c")
    tid = lax.axis_index("s")
    wid = cid * 16 + tid
    base = tid * _ROWS_PER_TILE

    idxv = (idx0, idx1, idx2)
    cfv = (cfb0, cfb1, cfb2)
    sciv = (sci0, sci1, sci2)
    rowsv = (rows0, rows1, rows2)
    semi = (semi0, semi1, semi2)
    semg = (semg0, semg1, semg2)
    sems = (sems0, sems1, sems2)

    zero16 = jnp.zeros((16,), jnp.float32)

    @plsc.parallel_loop(0, _CH)
    def _(j):
        for g in range(8):
            rows0[j, pl.ds(g * 16, 16)] = zero16

    nfull = _ROWS_PER_TILE // _CH
    rem = _ROWS_PER_TILE - nfull * _CH
    for k in range(nfull):
        pltpu.sync_copy(rows0, agg_sh.at[pl.ds(base + k * _CH, _CH)])
    if rem:
        pltpu.sync_copy(rows0.at[pl.ds(0, rem)],
                        agg_sh.at[pl.ds(base + nfull * _CH, rem)])
    plsc.subcore_barrier()

    def start_idx(c, b):
        d1 = pltpu.async_copy(rc_hbm.at[wid, c], idxv[b], semi[b])
        d2 = pltpu.async_copy(cf_hbm.at[wid, c], cfv[b], semi[b])
        return d1, d2

    def wait_idx(b):
        pltpu.make_async_copy(rc_hbm.at[wid, 0], idxv[b], semi[b]).wait()
        pltpu.make_async_copy(cf_hbm.at[wid, 0], cfv[b], semi[b]).wait()

    def start_gather(b):
        return pltpu.async_copy(hd_hbm.at[idxv[b].at[0]], rowsv[b], semg[b])

    def wait_gather(b):
        pltpu.make_async_copy(hd_hbm.at[idxv[b].at[0]], rowsv[b],
                              semg[b]).wait()

    def start_scatter(b):
        return pltpu.async_copy(rowsv[b], agg_sh.at[sciv[b].at[0]],
                                sems[b], add=True)

    def wait_scatter(b):
        pltpu.make_async_copy(rowsv[b], agg_sh.at[sciv[b].at[0]],
                              sems[b]).wait()

    def scale(b):
        @plsc.parallel_loop(0, _CH, unroll=4)
        def _(j):
            cj = plsc.load_gather(
                cfv[b],
                [jnp.broadcast_to(j, (16,)).astype(jnp.int32)])
            for g in range(8):
                sl = pl.ds(g * 16, 16)
                rowsv[b][j, sl] = rowsv[b][j, sl] * cj

    def copy_sci(b):
        for g in range(_CH // 16):
            sl = pl.ds(g * 16, 16)
            sciv[b][0, sl] = idxv[b][1, sl]

    def step(c, b, wait_s=True, start_g=True, start_i=True):
        b1 = (b + 1) % 3
        if wait_s:
            wait_scatter(b1)
        if start_g:
            wait_idx(b1)
            start_gather(b1)
        wait_gather(b)
        scale(b)
        copy_sci(b)
        start_scatter(b)
        if start_i:
            start_idx(c + 3, b)

    d = start_idx(0, 0)
    d[0].wait()
    d[1].wait()
    start_gather(0)
    start_idx(1, 1)
    start_idx(2, 2)
    step(0, 0, wait_s=False)
    step(1, 1, wait_s=False)

    def triple(k, _):
        c0 = 3 * k + 2
        step(c0, 2)
        step(c0 + 1, 0)
        step(c0 + 2, 1)
        return 0

    nk = (nch - 6) // 3
    lax.fori_loop(0, nk, triple, 0)

    cA = nch - 4
    step(cA, cA % 3)
    step(cA + 1, (cA + 1) % 3, start_i=False)
    step(cA + 2, (cA + 2) % 3, start_i=False)
    step(cA + 3, (cA + 3) % 3, start_g=False, start_i=False)
    wait_scatter((cA + 2) % 3)
    wait_scatter((cA + 3) % 3)

    plsc.subcore_barrier()
    pltpu.sync_copy(agg_sh.at[pl.ds(base, _ROWS_PER_TILE)],
                    part_hbm.at[cid, pl.ds(base, _ROWS_PER_TILE)])


def _sc_edge(hd, rc, cf):
    nch = rc.shape[1]
    f = pl.kernel(
        _sc_edge_body,
        out_type=jax.ShapeDtypeStruct((2, _N_SH, _D), jnp.float32),
        mesh=_MESH,
        scratch_types=[
            pltpu.VMEM((2, _CH), jnp.int32),
            pltpu.VMEM((2, _CH), jnp.int32),
            pltpu.VMEM((2, _CH), jnp.int32),
            pltpu.VMEM((_CH,), jnp.float32),
            pltpu.VMEM((_CH,), jnp.float32),
            pltpu.VMEM((_CH,), jnp.float32),
            pltpu.VMEM((1, _CH), jnp.int32),
            pltpu.VMEM((1, _CH), jnp.int32),
            pltpu.VMEM((1, _CH), jnp.int32),
            pltpu.VMEM((_CH, _D), jnp.float32),
            pltpu.VMEM((_CH, _D), jnp.float32),
            pltpu.VMEM((_CH, _D), jnp.float32),
            pltpu.VMEM_SHARED((_N_SH, _D), jnp.float32),
            pltpu.SemaphoreType.DMA,
            pltpu.SemaphoreType.DMA,
            pltpu.SemaphoreType.DMA,
            pltpu.SemaphoreType.DMA,
            pltpu.SemaphoreType.DMA,
            pltpu.SemaphoreType.DMA,
            pltpu.SemaphoreType.DMA,
            pltpu.SemaphoreType.DMA,
            pltpu.SemaphoreType.DMA,
        ],
        compiler_params=_SC_PARAMS,
    )
    return f(hd, rc, cf)


def kernel(x, edge_index, att_l, att_r, ln_gamma, ln_beta, W, b):
    n, d = x.shape
    e = edge_index.shape[1]
    row = edge_index[0]
    col = edge_index[1]
    xp = jnp.pad(x, ((0, _NP - n), (0, 0)))
    mask = (jnp.arange(_NP) < n).astype(jnp.float32)

    epw = e // _NW
    nch = (epw + _CH - 1) // _CH
    nch = ((nch + 2) // 3) * 3
    pad = nch * _CH - epw
    row3 = jnp.pad(row.reshape(_NW, epw), ((0, 0), (0, pad))) \
        .reshape(_NW, nch, 1, _CH)
    col3 = jnp.pad(col.reshape(_NW, epw), ((0, 0), (0, pad)),
                   constant_values=_DUMMY).reshape(_NW, nch, 1, _CH)
    rc = jnp.concatenate([row3, col3], axis=2)

    degp = _sc_deg(rc)
    histp = jnp.pad(degp, ((0, 0), (0, _NP - _N_SH)))
    dis, al, ar, hd = _tc_prep(histp, mask, xp, att_l[0], att_r[0])

    h = xp
    for layer in range(N_LAYERS):
        cf = _sc_coef(rc, al[:_N_SH], ar[:_N_SH])
        part = _sc_edge(hd, rc, cf)
        p0 = jnp.pad(part[0], ((0, _NP - _N_SH), (0, 0)))
        p1 = jnp.pad(part[1], ((0, _NP - _N_SH), (0, 0)))
        if layer < N_LAYERS - 1:
            h, al, ar, hd = _tc_dense(p0, p1, h, xp, al, ar, dis,
                                      ln_gamma[layer], ln_beta[layer],
                                      att_l[layer + 1], att_r[layer + 1])
        else:
            out = _tc_final(p0, p1, h, xp, al, ar, dis,
                            ln_gamma[layer], ln_beta[layer], W, b)
    return out[:n]


# speedup vs baseline: 2.3188x; 1.0546x over previous
import functools

import jax
import jax.numpy as jnp
from jax import lax
from jax.experimental import pallas as pl
from jax.experimental.pallas import tpu as pltpu
from jax.experimental.pallas import tpu_sc as plsc

N_LAYERS = 4
EPS_FA = 1.0
LN_EPS = 1e-5

_BLK = 1024
_NW = 32
_CH = 112
_D = 128

_NP = 10240
_N_SH = 10016
_ROWS_PER_TILE = _N_SH // 16
_DUMMY = 10000


def _elu(x):
    return jnp.where(x > 0, x, jnp.exp(jnp.minimum(x, 0.0)) - 1.0)


def _tc_prep_body(histp_ref, mask_ref, x_ref, attl_ref, attr_ref,
                  dis_ref, al_ref, ar_ref, hd_ref):
    deg = jnp.sum(histp_ref[...], axis=0) + 1.0
    dis = mask_ref[...] * jax.lax.rsqrt(deg)
    dis_ref[...] = dis
    x = x_ref[...]
    al_ref[...] = jnp.sum(x * attl_ref[...][None, :], axis=1)
    ar_ref[...] = jnp.sum(x * attr_ref[...][None, :], axis=1)
    hd_ref[...] = dis[:, None] * x


def _tc_prep(histp, mask, x, attl, attr):
    np_, d = x.shape
    grid = np_ // _BLK
    v1 = pl.BlockSpec((_BLK,), lambda i: (i,))
    vfull = pl.BlockSpec((d,), lambda i: (0,))
    m2 = pl.BlockSpec((_BLK, d), lambda i: (i, 0))
    hp = pl.BlockSpec((_NW, _BLK), lambda i: (0, i))
    return pl.pallas_call(
        _tc_prep_body,
        grid=(grid,),
        in_specs=[hp, v1, m2, vfull, vfull],
        out_specs=[v1, v1, v1, m2],
        out_shape=[jax.ShapeDtypeStruct((np_,), jnp.float32),
                   jax.ShapeDtypeStruct((np_,), jnp.float32),
                   jax.ShapeDtypeStruct((np_,), jnp.float32),
                   jax.ShapeDtypeStruct((np_, d), jnp.float32)],
    )(histp, mask, x, attl, attr)


def _tc_dense_body(p0_ref, p1_ref, h_ref, h0_ref, al_ref, ar_ref, dis_ref,
                   g_ref, b_ref, attl_ref, attr_ref,
                   hn_ref, aln_ref, arn_ref, hd_ref):
    dis = dis_ref[...]
    selfc = jnp.tanh(al_ref[...] + ar_ref[...]) * dis * dis
    agg = (dis[:, None] * (p0_ref[...] + p1_ref[...])
           + selfc[:, None] * h_ref[...] + EPS_FA * h0_ref[...])
    a = _elu(agg)
    mu = jnp.mean(a, axis=1, keepdims=True)
    var = jnp.mean((a - mu) ** 2, axis=1, keepdims=True)
    hn = (a - mu) * jax.lax.rsqrt(var + LN_EPS) * g_ref[...][None, :] \
        + b_ref[...][None, :]
    hn_ref[...] = hn
    aln_ref[...] = jnp.sum(hn * attl_ref[...][None, :], axis=1)
    arn_ref[...] = jnp.sum(hn * attr_ref[...][None, :], axis=1)
    hd_ref[...] = dis[:, None] * hn


def _tc_dense(p0, p1, h, h0, al, ar, dis, gamma, beta, attl, attr):
    np_, d = h.shape
    grid = np_ // _BLK
    v1 = pl.BlockSpec((_BLK,), lambda i: (i,))
    vfull = pl.BlockSpec((d,), lambda i: (0,))
    m2 = pl.BlockSpec((_BLK, d), lambda i: (i, 0))
    return pl.pallas_call(
        _tc_dense_body,
        grid=(grid,),
        in_specs=[m2, m2, m2, m2, v1, v1, v1, vfull, vfull, vfull, vfull],
        out_specs=[m2, v1, v1, m2],
        out_shape=[jax.ShapeDtypeStruct((np_, d), jnp.float32),
                   jax.ShapeDtypeStruct((np_,), jnp.float32),
                   jax.ShapeDtypeStruct((np_,), jnp.float32),
                   jax.ShapeDtypeStruct((np_, d), jnp.float32)],
    )(p0, p1, h, h0, al, ar, dis, gamma, beta, attl, attr)


def _tc_final_body(p0_ref, p1_ref, h_ref, h0_ref, al_ref, ar_ref, dis_ref,
                   g_ref, b_ref, w_ref, bias_ref, out_ref):
    dis = dis_ref[...]
    selfc = jnp.tanh(al_ref[...] + ar_ref[...]) * dis * dis
    agg = (dis[:, None] * (p0_ref[...] + p1_ref[...])
           + selfc[:, None] * h_ref[...] + EPS_FA * h0_ref[...])
    a = _elu(agg)
    mu = jnp.mean(a, axis=1, keepdims=True)
    var = jnp.mean((a - mu) ** 2, axis=1, keepdims=True)
    hn = (a - mu) * jax.lax.rsqrt(var + LN_EPS) * g_ref[...][None, :] \
        + b_ref[...][None, :]
    out_ref[...] = jax.lax.dot_general(
        hn, w_ref[...], (((1,), (1,)), ((), ())),
        preferred_element_type=jnp.float32) + bias_ref[...][None, :]


def _tc_final(p0, p1, h, h0, al, ar, dis, gamma, beta, w, bias):
    np_, d = h.shape
    c = w.shape[0]
    grid = np_ // _BLK
    v1 = pl.BlockSpec((_BLK,), lambda i: (i,))
    vfull = pl.BlockSpec((d,), lambda i: (0,))
    m2 = pl.BlockSpec((_BLK, d), lambda i: (i, 0))
    wspec = pl.BlockSpec((c, d), lambda i: (0, 0))
    bspec = pl.BlockSpec((c,), lambda i: (0,))
    return pl.pallas_call(
        _tc_final_body,
        grid=(grid,),
        in_specs=[m2, m2, m2, m2, v1, v1, v1, vfull, vfull, wspec, bspec],
        out_specs=pl.BlockSpec((_BLK, c), lambda i: (i, 0)),
        out_shape=jax.ShapeDtypeStruct((np_, c), jnp.float32),
    )(p0, p1, h, h0, al, ar, dis, gamma, beta, w, bias)


_MESH = plsc.VectorSubcoreMesh(core_axis_name="c", subcore_axis_name="s")
_SC_PARAMS = pltpu.CompilerParams(
    needs_layout_passes=False, use_tc_tiling_on_sc=False)


def _sc_deg_body(rc_hbm, deg_hbm, rc_v, deg_v):
    nch = rc_hbm.shape[1]
    cid = lax.axis_index("c")
    tid = lax.axis_index("s")
    wid = cid * 16 + tid
    pltpu.sync_copy(rc_hbm.at[wid], rc_v)

    zero16 = jnp.zeros((16,), jnp.float32)

    def zrow(j, _):
        deg_v[pl.ds(j * 16, 16)] = zero16
        return 0

    lax.fori_loop(0, deg_v.shape[0] // 16, zrow, 0)

    ones16 = jnp.ones((16,), jnp.float32)

    def chunk(ch, _):
        for g in range(_CH // 16):
            c = rc_v[ch, 1, pl.ds(g * 16, 16)]
            plsc.addupdate_scatter(deg_v, [c], ones16)
        return 0

    lax.fori_loop(0, nch, chunk, 0)
    pltpu.sync_copy(deg_v, deg_hbm.at[wid])


def _sc_deg(rc):
    nch = rc.shape[1]
    f = pl.kernel(
        _sc_deg_body,
        out_type=jax.ShapeDtypeStruct((_NW, _N_SH), jnp.float32),
        mesh=_MESH,
        scratch_types=[
            pltpu.VMEM((nch, 2, _CH), jnp.int32),
            pltpu.VMEM((_N_SH,), jnp.float32),
        ],
        compiler_params=_SC_PARAMS,
    )
    return f(rc)


def _sc_coef_body(rc_hbm, al_hbm, ar_hbm, coef_hbm, rc_v, al_v, ar_v,
                  coef_v):
    nch = rc_hbm.shape[1]
    cid = lax.axis_index("c")
    tid = lax.axis_index("s")
    wid = cid * 16 + tid
    pltpu.sync_copy(rc_hbm.at[wid], rc_v)
    pltpu.sync_copy(al_hbm, al_v)
    pltpu.sync_copy(ar_hbm, ar_v)

    @plsc.parallel_loop(0, nch, unroll=2)
    def _(ch):
        for g in range(_CH // 16):
            sl = pl.ds(g * 16, 16)
            r = rc_v[ch, 0, sl]
            c = rc_v[ch, 1, sl]
            s = plsc.load_gather(al_v, [r]) + plsc.load_gather(ar_v, [c])
            e = jnp.exp(-2.0 * jnp.abs(s))
            t = (1.0 - e) / (1.0 + e)
            coef_v[ch, sl] = jnp.where(s < 0, -t, t)

    pltpu.sync_copy(coef_v, coef_hbm.at[wid])


def _sc_coef(rc, al_p, ar_p):
    nch = rc.shape[1]
    f = pl.kernel(
        _sc_coef_body,
        out_type=jax.ShapeDtypeStruct((_NW, nch, _CH), jnp.float32),
        mesh=_MESH,
        scratch_types=[
            pltpu.VMEM((nch, 2, _CH), jnp.int32),
            pltpu.VMEM((_N_SH,), jnp.float32),
            pltpu.VMEM((_N_SH,), jnp.float32),
            pltpu.VMEM((nch, _CH), jnp.float32),
        ],
        compiler_params=_SC_PARAMS,
    )
    return f(rc, al_p, ar_p)


def _sc_edge_body(hd_hbm, rc_hbm, cf_hbm, part_hbm,
                  idx0, idx1, idx2, cfb0, cfb1, cfb2, sci0, sci1, sci2,
                  rows0, rows1, rows2, agg_sh,
                  semi0, semi1, semi2, semg0, semg1, semg2,
                  sems0, sems1, sems2):
    nch = rc_hbm.shape[1]
    cid = lax.axis_index("c")
    tid = lax.axis_index("s")
    wid = cid * 16 + tid
    base = tid * _ROWS_PER_TILE

    idxv = (idx0, idx1, idx2)
    cfv = (cfb0, cfb1, cfb2)
    sciv = (sci0, sci1, sci2)
    rowsv = (rows0, rows1, rows2)
    semi = (semi0, semi1, semi2)
    semg = (semg0, semg1, semg2)
    sems = (sems0, sems1, sems2)

    zero16 = jnp.zeros((16,), jnp.float32)

    @plsc.parallel_loop(0, _CH)
    def _(j):
        for g in range(8):
            rows0[j, pl.ds(g * 16, 16)] = zero16

    nfull = _ROWS_PER_TILE // _CH
    rem = _ROWS_PER_TILE - nfull * _CH
    for k in range(nfull):
        pltpu.sync_copy(rows0, agg_sh.at[pl.ds(base + k * _CH, _CH)])
    if rem:
        pltpu.sync_copy(rows0.at[pl.ds(0, rem)],
                        agg_sh.at[pl.ds(base + nfull * _CH, rem)])
    plsc.subcore_barrier()

    def start_idx(c, b):
        d1 = pltpu.async_copy(rc_hbm.at[wid, c], idxv[b], semi[b])
        d2 = pltpu.async_copy(cf_hbm.at[wid, c], cfv[b], semi[b])
        return d1, d2

    def wait_idx(b):
        pltpu.make_async_copy(rc_hbm.at[wid, 0], idxv[b], semi[b]).wait()
        pltpu.make_async_copy(cf_hbm.at[wid, 0], cfv[b], semi[b]).wait()

    def start_gather(b):
        return pltpu.async_copy(hd_hbm.at[idxv[b].at[0]], rowsv[b], semg[b])

    def wait_gather(b):
        pltpu.make_async_copy(hd_hbm.at[idxv[b].at[0]], rowsv[b],
                              semg[b]).wait()

    def start_scatter(b):
        return pltpu.async_copy(rowsv[b], agg_sh.at[sciv[b].at[0]],
                                sems[b], add=True)

    def wait_scatter(b):
        pltpu.make_async_copy(rowsv[b], agg_sh.at[sciv[b].at[0]],
                              sems[b]).wait()

    def scale(b):
        @plsc.parallel_loop(0, _CH, unroll=4)
        def _(j):
            cj = plsc.load_gather(
                cfv[b],
                [jnp.broadcast_to(j, (16,)).astype(jnp.int32)])
            for g in range(8):
                sl = pl.ds(g * 16, 16)
                rowsv[b][j, sl] = rowsv[b][j, sl] * cj

    def copy_sci(b):
        for g in range(_CH // 16):
            sl = pl.ds(g * 16, 16)
            sciv[b][0, sl] = idxv[b][1, sl]

    def step(c, b, wait_s=True, start_g=True, start_i=True):
        b1 = (b + 1) % 3
        if wait_s:
            wait_scatter(b1)
        if start_g:
            wait_idx(b1)
            start_gather(b1)
        wait_gather(b)
        scale(b)
        copy_sci(b)
        start_scatter(b)
        if start_i:
            start_idx(c + 3, b)

    d = start_idx(0, 0)
    d[0].wait()
    d[1].wait()
    start_gather(0)
    start_idx(1, 1)
    start_idx(2, 2)
    step(0, 0, wait_s=False)
    step(1, 1, wait_s=False)

    def triple(k, _):
        c0 = 3 * k + 2
        step(c0, 2)
        step(c0 + 1, 0)
        step(c0 + 2, 1)
        return 0

    nk = (nch - 6) // 3
    lax.fori_loop(0, nk, triple, 0)

    cA = nch - 4
    step(cA, cA % 3)
    step(cA + 1, (cA + 1) % 3, start_i=False)
    step(cA + 2, (cA + 2) % 3, start_i=False)
    step(cA + 3, (cA + 3) % 3, start_g=False, start_i=False)
    wait_scatter((cA + 2) % 3)
    wait_scatter((cA + 3) % 3)

    plsc.subcore_barrier()
    pltpu.sync_copy(agg_sh.at[pl.ds(base, _ROWS_PER_TILE)],
                    part_hbm.at[cid, pl.ds(base, _ROWS_PER_TILE)])


def _sc_edge(hd, rc, cf):
    nch = rc.shape[1]
    f = pl.kernel(
        _sc_edge_body,
        out_type=jax.ShapeDtypeStruct((2, _N_SH, _D), jnp.float32),
        mesh=_MESH,
        scratch_types=[
            pltpu.VMEM((2, _CH), jnp.int32),
            pltpu.VMEM((2, _CH), jnp.int32),
            pltpu.VMEM((2, _CH), jnp.int32),
            pltpu.VMEM((_CH,), jnp.float32),
            pltpu.VMEM((_CH,), jnp.float32),
            pltpu.VMEM((_CH,), jnp.float32),
            pltpu.VMEM((1, _CH), jnp.int32),
            pltpu.VMEM((1, _CH), jnp.int32),
            pltpu.VMEM((1, _CH), jnp.int32),
            pltpu.VMEM((_CH, _D), jnp.float32),
            pltpu.VMEM((_CH, _D), jnp.float32),
            pltpu.VMEM((_CH, _D), jnp.float32),
            pltpu.VMEM_SHARED((_N_SH, _D), jnp.float32),
            pltpu.SemaphoreType.DMA,
            pltpu.SemaphoreType.DMA,
            pltpu.SemaphoreType.DMA,
            pltpu.SemaphoreType.DMA,
            pltpu.SemaphoreType.DMA,
            pltpu.SemaphoreType.DMA,
            pltpu.SemaphoreType.DMA,
            pltpu.SemaphoreType.DMA,
            pltpu.SemaphoreType.DMA,
        ],
        compiler_params=_SC_PARAMS,
    )
    return f(hd, rc, cf)


def kernel(x, edge_index, att_l, att_r, ln_gamma, ln_beta, W, b):
    n, d = x.shape
    e = edge_index.shape[1]
    row = edge_index[0]
    col = edge_index[1]
    xp = jnp.pad(x, ((0, _NP - n), (0, 0)))
    mask = (jnp.arange(_NP) < n).astype(jnp.float32)

    epw = e // _NW
    nch = (epw + _CH - 1) // _CH
    nch = ((nch + 2) // 3) * 3
    pad = nch * _CH - epw
    row3 = jnp.pad(row.reshape(_NW, epw), ((0, 0), (0, pad))) \
        .reshape(_NW, nch, 1, _CH)
    col3 = jnp.pad(col.reshape(_NW, epw), ((0, 0), (0, pad)),
                   constant_values=_DUMMY).reshape(_NW, nch, 1, _CH)
    rc = jnp.concatenate([row3, col3], axis=2)

    degp = _sc_deg(rc)
    histp = jnp.pad(degp, ((0, 0), (0, _NP - _N_SH)))
    dis, al, ar, hd = _tc_prep(histp, mask, xp, att_l[0], att_r[0])

    h = xp
    for layer in range(N_LAYERS):
        cf = _sc_coef(rc, al[:_N_SH], ar[:_N_SH])
        part = _sc_edge(hd, rc, cf)
        p0 = jnp.pad(part[0], ((0, _NP - _N_SH), (0, 0)))
        p1 = jnp.pad(part[1], ((0, _NP - _N_SH), (0, 0)))
        if layer < N_LAYERS - 1:
            h, al, ar, hd = _tc_dense(p0, p1, h, xp, al, ar, dis,
                                      ln_gamma[layer], ln_beta[layer],
                                      att_l[layer + 1], att_r[layer + 1])
        else:
            out = _tc_final(p0, p1, h, xp, al, ar, dis,
                            ln_gamma[layer], ln_beta[layer], W, b)
    return out[:n]
